# baseline (speedup 1.0000x reference)
import jax
import jax.numpy as jnp
from jax.experimental import pallas as pl

_SDF = 128
_NPTS = 16


def _raster_body(ys_ref, xs_ref, res_ref, org_ref, out_ref):
    y = ys_ref[0]
    xx = xs_ref[0]
    res = res_ref[0]
    org = org_ref[0]
    row = (y / res[:, 0:1] + org[:, 0:1]).astype(jnp.int32)
    col = (xx / res[:, 1:2] + org[:, 1:2]).astype(jnp.int32)
    row = jnp.clip(row, 0, _SDF - 1)
    col = jnp.clip(col, 0, _SDF - 1)
    ri = jax.lax.broadcasted_iota(jnp.int32, (_SDF, _SDF, _NPTS), 0)
    ci = jax.lax.broadcasted_iota(jnp.int32, (_SDF, _SDF, _NPTS), 1)
    mask = (ri == row.reshape(1, 1, _NPTS)) & (ci == col.reshape(1, 1, _NPTS))
    out_ref[0] = mask.astype(jnp.float32)


def kernel(x, resolution, origin):
    b = x.shape[0]
    pts = x.reshape(b, _NPTS, 2)
    ys = pts[:, :, 1].reshape(b, 1, _NPTS)
    xs = pts[:, :, 0].reshape(b, 1, _NPTS)
    res3 = resolution.reshape(b, 1, 2)
    org3 = origin.reshape(b, 1, 2)
    return pl.pallas_call(
        _raster_body,
        grid=(b,),
        in_specs=[
            pl.BlockSpec((1, 1, _NPTS), lambda i: (i, 0, 0)),
            pl.BlockSpec((1, 1, _NPTS), lambda i: (i, 0, 0)),
            pl.BlockSpec((1, 1, 2), lambda i: (i, 0, 0)),
            pl.BlockSpec((1, 1, 2), lambda i: (i, 0, 0)),
        ],
        out_specs=pl.BlockSpec((1, _SDF, _SDF, _NPTS), lambda i: (i, 0, 0, 0)),
        out_shape=jax.ShapeDtypeStruct((b, _SDF, _SDF, _NPTS), jnp.float32),
    )(ys, xs, res3, org3)


# speedup vs baseline: 2.9803x; 2.9803x over previous
import jax
import jax.numpy as jnp
from jax.experimental import pallas as pl

_SDF = 128
_NPTS = 16
_LANES = _SDF * _NPTS


def _raster_body(yt_ref, xt_ref, res_ref, org_ref, out_ref):
    y = yt_ref[0]
    xx = xt_ref[0]
    res = res_ref[0]
    org = org_ref[0]
    row = jnp.clip((y / res[:, 0:1] + org[:, 0:1]).astype(jnp.int32), 0, _SDF - 1)
    col = jnp.clip((xx / res[:, 1:2] + org[:, 1:2]).astype(jnp.int32), 0, _SDF - 1)
    lane = jax.lax.broadcasted_iota(jnp.int32, (1, _LANES), 1)
    key = jnp.where(col == (lane >> 4), row, -1)
    ri = jax.lax.broadcasted_iota(jnp.int32, (_SDF, _LANES), 0)
    out_ref[0] = (ri == key).astype(jnp.float32)


def kernel(x, resolution, origin):
    b = x.shape[0]
    pts = x.reshape(b, _NPTS, 2)
    ys = jnp.tile(pts[:, :, 1], (1, _SDF)).reshape(b, 1, _LANES)
    xs = jnp.tile(pts[:, :, 0], (1, _SDF)).reshape(b, 1, _LANES)
    res3 = resolution.reshape(b, 1, 2)
    org3 = origin.reshape(b, 1, 2)
    out = pl.pallas_call(
        _raster_body,
        grid=(b,),
        in_specs=[
            pl.BlockSpec((1, 1, _LANES), lambda i: (i, 0, 0)),
            pl.BlockSpec((1, 1, _LANES), lambda i: (i, 0, 0)),
            pl.BlockSpec((1, 1, 2), lambda i: (i, 0, 0)),
            pl.BlockSpec((1, 1, 2), lambda i: (i, 0, 0)),
        ],
        out_specs=pl.BlockSpec((1, _SDF, _LANES), lambda i: (i, 0, 0)),
        out_shape=jax.ShapeDtypeStruct((b, _SDF, _LANES), jnp.float32),
    )(ys, xs, res3, org3)
    return out.reshape(b, _SDF, _SDF, _NPTS)
